# baseline (speedup 1.0000x reference)
import jax
import jax.numpy as jnp
from jax.experimental import pallas as pl
from jax.experimental.pallas import tpu as pltpu

_BM = 512
_WC = 256


def _linear_block_kernel(x_ref, w_hbm_ref, b_ref, o_ref,
                         wb_ref, stage_ref, sems):
    n_chunks = stage_ref.shape[0]
    i = pl.program_id(0)

    @pl.when(i == 0)
    def _():
        for j in range(n_chunks):
            pltpu.make_async_copy(
                w_hbm_ref.at[:, pl.ds(j * _WC, _WC)],
                stage_ref.at[j],
                sems.at[j],
            ).start()
        xb = x_ref[...].astype(jnp.bfloat16)
        for j in range(n_chunks):
            pltpu.make_async_copy(
                stage_ref.at[j], stage_ref.at[j], sems.at[j]
            ).wait()
            wc = stage_ref[j].astype(jnp.bfloat16)
            wb_ref[:, pl.ds(j * _WC, _WC)] = wc
            o_ref[:, pl.ds(j * _WC, _WC)] = (
                jnp.dot(xb, wc, preferred_element_type=jnp.float32)
                + b_ref[:, pl.ds(j * _WC, _WC)]
            )

    @pl.when(i != 0)
    def _():
        xb = x_ref[...].astype(jnp.bfloat16)
        o_ref[...] = (
            jnp.dot(xb, wb_ref[...], preferred_element_type=jnp.float32)
            + b_ref[...]
        )


def kernel(x, w_t, bias):
    B, F_in = x.shape
    F_out = w_t.shape[1]
    bm = min(_BM, B)
    assert B % bm == 0, "row count must tile evenly"
    assert F_out % _WC == 0, "feature count must tile evenly"
    n_chunks = F_out // _WC

    b_row = bias.astype(jnp.float32).reshape(1, F_out)

    return pl.pallas_call(
        _linear_block_kernel,
        out_shape=jax.ShapeDtypeStruct((B, F_out), x.dtype),
        grid=(B // bm,),
        in_specs=[
            pl.BlockSpec((bm, F_in), lambda i: (i, 0)),
            pl.BlockSpec(memory_space=pl.ANY),
            pl.BlockSpec((1, F_out), lambda i: (0, 0)),
        ],
        out_specs=pl.BlockSpec((bm, F_out), lambda i: (i, 0)),
        scratch_shapes=[
            pltpu.VMEM((F_in, F_out), jnp.bfloat16),
            pltpu.VMEM((n_chunks, F_in, _WC), jnp.float32),
            pltpu.SemaphoreType.DMA((n_chunks,)),
        ],
        compiler_params=pltpu.CompilerParams(
            dimension_semantics=("arbitrary",),
            vmem_limit_bytes=60 << 20,
        ),
    )(x, w_t, b_row)


# speedup vs baseline: 1.0182x; 1.0182x over previous
import jax
import jax.numpy as jnp
from jax.experimental import pallas as pl
from jax.experimental.pallas import tpu as pltpu

_BM = 512
_WR = 256


def _linear_block_kernel(x_ref, w_hbm_ref, b_ref, o_ref,
                         wb_ref, stage_ref, sems):
    n_chunks = stage_ref.shape[0]
    i = pl.program_id(0)

    @pl.when(i == 0)
    def _():
        for k in range(n_chunks):
            pltpu.make_async_copy(
                w_hbm_ref.at[pl.ds(k * _WR, _WR), :],
                stage_ref.at[k],
                sems.at[k],
            ).start()
        for k in range(n_chunks):
            pltpu.make_async_copy(
                stage_ref.at[k], stage_ref.at[k], sems.at[k]
            ).wait()
            wb_ref[pl.ds(k * _WR, _WR), :] = stage_ref[k].astype(jnp.bfloat16)

    @pl.when(i > 0)
    def _():
        xb = x_ref[...].astype(jnp.bfloat16)
        o_ref[...] = (
            jnp.dot(xb, wb_ref[...], preferred_element_type=jnp.float32)
            + b_ref[...]
        )


def kernel(x, w_t, bias):
    B, F_in = x.shape
    F_out = w_t.shape[1]
    bm = min(_BM, B)
    assert B % bm == 0, "row count must tile evenly"
    assert F_in % _WR == 0, "contraction dim must tile evenly"
    n_chunks = F_in // _WR

    b_row = bias.astype(jnp.float32).reshape(1, F_out)

    def _shifted(i):
        return (jax.lax.max(i - 1, 0), 0)

    return pl.pallas_call(
        _linear_block_kernel,
        out_shape=jax.ShapeDtypeStruct((B, F_out), x.dtype),
        grid=(B // bm + 1,),
        in_specs=[
            pl.BlockSpec((bm, F_in), _shifted),
            pl.BlockSpec(memory_space=pl.ANY),
            pl.BlockSpec((1, F_out), lambda i: (0, 0)),
        ],
        out_specs=pl.BlockSpec((bm, F_out), _shifted),
        scratch_shapes=[
            pltpu.VMEM((F_in, F_out), jnp.bfloat16),
            pltpu.VMEM((n_chunks, _WR, F_out), jnp.float32),
            pltpu.SemaphoreType.DMA((n_chunks,)),
        ],
        compiler_params=pltpu.CompilerParams(
            dimension_semantics=("arbitrary",),
            vmem_limit_bytes=60 << 20,
        ),
    )(x, w_t, b_row)


# speedup vs baseline: 1.0211x; 1.0028x over previous
import jax
import jax.numpy as jnp
from jax.experimental import pallas as pl
from jax.experimental.pallas import tpu as pltpu

_BM = 512


def _linear_block_kernel(x_ref, w_ref, b_ref, o_ref, wb_ref):
    @pl.when(pl.program_id(0) == 0)
    def _():
        wb_ref[...] = w_ref[...].astype(jnp.bfloat16)

    xb = x_ref[...].astype(jnp.bfloat16)
    acc = jnp.dot(xb, wb_ref[...], preferred_element_type=jnp.float32)
    o_ref[...] = acc + b_ref[...]


def kernel(x, w_t, bias):
    B, F_in = x.shape
    F_out = w_t.shape[1]
    bm = min(_BM, B)
    assert B % bm == 0, "row count must tile evenly"

    b_row = bias.astype(jnp.float32).reshape(1, F_out)

    return pl.pallas_call(
        _linear_block_kernel,
        out_shape=jax.ShapeDtypeStruct((B, F_out), x.dtype),
        grid=(B // bm,),
        in_specs=[
            pl.BlockSpec((bm, F_in), lambda i: (i, 0)),
            pl.BlockSpec((F_in, F_out), lambda i: (0, 0)),
            pl.BlockSpec((1, F_out), lambda i: (0, 0)),
        ],
        out_specs=pl.BlockSpec((bm, F_out), lambda i: (i, 0)),
        scratch_shapes=[pltpu.VMEM((F_in, F_out), jnp.bfloat16)],
        compiler_params=pltpu.CompilerParams(
            dimension_semantics=("arbitrary",),
            vmem_limit_bytes=60 << 20,
        ),
    )(x, w_t, b_row)
